# Initial kernel scaffold; baseline (speedup 1.0000x reference)
import jax
import jax.numpy as jnp
from jax.experimental import pallas as pl


def kernel(x, edge_index, batch, W1, b1, W2, b2, W3, b3, Wl, bl):
    raise NotImplementedError("write your pallas kernel here")


# speedup vs baseline: 22.6312x; 22.6312x over previous
import functools

import jax
import jax.numpy as jnp
from jax import lax
from jax.experimental import pallas as pl
from jax.experimental.pallas import tpu as pltpu
from jax.experimental.pallas import tpu_sc as plsc

N = 10000
E = 320000
G = 64
NC = 2
NS = 16
NW = NC * NS
EPT = E // NW
CHUNK = 80
NCHUNK = EPT // CHUNK
RPT = 624
TAIL = N - NS * RPT
DEGW = 16

@functools.cache
def _mesh():
    return plsc.VectorSubcoreMesh(core_axis_name="c", subcore_axis_name="s",
                                  num_cores=NC, num_subcores=NS)


def _zero_slice(zeros_hbm, shared, s):
    row0 = s * RPT
    pltpu.sync_copy(zeros_hbm.at[pl.ds(row0, RPT)], shared.at[pl.ds(row0, RPT)])

    @pl.when(s == NS - 1)
    def _():
        pltpu.sync_copy(zeros_hbm.at[pl.ds(NS * RPT, TAIL)],
                        shared.at[pl.ds(NS * RPT, TAIL)])


def _write_slice(shared, out_hbm, c, s):
    row0 = s * RPT
    pltpu.sync_copy(shared.at[pl.ds(row0, RPT)],
                    out_hbm.at[c, pl.ds(row0, RPT)])

    @pl.when(s == NS - 1)
    def _():
        pltpu.sync_copy(shared.at[pl.ds(NS * RPT, TAIL)],
                        out_hbm.at[c, pl.ds(NS * RPT, TAIL)])


def _deg_body(dst_hbm, ones_hbm, zeros_hbm, out_hbm, idx_v, ones_v, shared):
    c = lax.axis_index("c")
    s = lax.axis_index("s")
    w = c * NS + s
    _zero_slice(zeros_hbm, shared, s)
    pltpu.sync_copy(dst_hbm.at[w], idx_v)
    pltpu.sync_copy(ones_hbm, ones_v)
    plsc.subcore_barrier()

    def body(j, carry):
        pltpu.sync_copy(ones_v, shared.at[idx_v.at[j]], add=True)
        return carry

    lax.fori_loop(0, NCHUNK, body, 0)
    plsc.subcore_barrier()
    _write_slice(shared, out_hbm, c, s)


_SC_PARAMS = pltpu.CompilerParams(use_tc_tiling_on_sc=False)


@functools.cache
def _deg_kernel():
    return pl.kernel(
        _deg_body,
        out_type=jax.ShapeDtypeStruct((NC, N, DEGW), jnp.float32),
        mesh=_mesh(),
        compiler_params=_SC_PARAMS,
        scratch_types=[
            pltpu.VMEM((NCHUNK, CHUNK), jnp.int32),
            pltpu.VMEM((CHUNK, DEGW), jnp.float32),
            pltpu.VMEM_SHARED((N, DEGW), jnp.float32),
        ],
    )


def _conv_body(h_hbm, src_hbm, dst_hbm, zeros_hbm, out_hbm,
               src_v, dst_v, rows_v, shared, sem):
    c = lax.axis_index("c")
    s = lax.axis_index("s")
    w = c * NS + s
    _zero_slice(zeros_hbm, shared, s)
    pltpu.sync_copy(src_hbm.at[w], src_v)
    pltpu.sync_copy(dst_hbm.at[w], dst_v)
    plsc.subcore_barrier()

    def body(j, carry):
        pltpu.async_copy(h_hbm.at[src_v.at[j]], rows_v, sem).wait()
        pltpu.sync_copy(rows_v, shared.at[dst_v.at[j]], add=True)
        return carry

    lax.fori_loop(0, NCHUNK, body, 0)
    plsc.subcore_barrier()
    _write_slice(shared, out_hbm, c, s)


@functools.cache
def _make_conv(d):
    return pl.kernel(
        _conv_body,
        out_type=jax.ShapeDtypeStruct((NC, N, d), jnp.float32),
        mesh=_mesh(),
        compiler_params=_SC_PARAMS,
        scratch_types=[
            pltpu.VMEM((NCHUNK, CHUNK), jnp.int32),
            pltpu.VMEM((NCHUNK, CHUNK), jnp.int32),
            pltpu.VMEM((CHUNK, d), jnp.float32),
            pltpu.VMEM_SHARED((N, d), jnp.float32),
            pltpu.SemaphoreType.DMA,
        ],
    )


def _dis(degp_ref):
    return lax.rsqrt(degp_ref[0, :, 0:1] + degp_ref[1, :, 0:1] + 1.0)


def _tc0_body(degp_ref, x_ref, w_ref, o_ref):
    o_ref[...] = jnp.dot(x_ref[...], w_ref[...],
                         preferred_element_type=jnp.float32) * _dis(degp_ref)


def _tcmid_body(degp_ref, p_ref, z_ref, b_ref, w_ref, o_ref):
    dis = _dis(degp_ref)
    u = dis * (p_ref[0] + p_ref[1] + z_ref[...]) + b_ref[...]
    h = jnp.maximum(u, 0.0)
    o_ref[...] = jnp.dot(h, w_ref[...],
                         preferred_element_type=jnp.float32) * dis


def _tc3_body(degp_ref, p_ref, z_ref, b_ref, batch_ref, wl_ref, bl_ref, o_ref):
    dis = _dis(degp_ref)
    h3 = dis * (p_ref[0] + p_ref[1] + z_ref[...]) + b_ref[...]
    bt = batch_ref[...]
    seg = lax.broadcasted_iota(jnp.int32, (G, 1), 0)
    onehot = (bt == seg).astype(jnp.float32)
    sums = jnp.dot(onehot, h3, preferred_element_type=jnp.float32)
    cnt = jnp.sum(onehot, axis=1, keepdims=True)
    pooled = sums / jnp.maximum(cnt, 1.0)
    o_ref[...] = jnp.dot(pooled, wl_ref[...],
                         preferred_element_type=jnp.float32) + bl_ref[...]


def _tc(body, out_shape):
    return pl.pallas_call(body, out_shape=jax.ShapeDtypeStruct(out_shape,
                                                               jnp.float32))


_tc0 = _tc(_tc0_body, (N, 64))
_tcmid64 = _tc(_tcmid_body, (N, 32))
_tcmid32 = _tc(_tcmid_body, (N, 32))
_tc3 = _tc(_tc3_body, (G, 10))


def kernel(x, edge_index, batch, W1, b1, W2, b2, W3, b3, Wl, bl):
    src = edge_index[0].reshape(NW, NCHUNK, CHUNK)
    dst = edge_index[1].reshape(NW, NCHUNK, CHUNK)
    ones16 = jnp.ones((CHUNK, DEGW), jnp.float32)
    zeros16 = jnp.zeros((N, DEGW), jnp.float32)
    zeros64 = jnp.zeros((N, 64), jnp.float32)
    zeros32 = jnp.zeros((N, 32), jnp.float32)

    degp = _deg_kernel()(dst, ones16, zeros16)
    z1 = _tc0(degp, x, W1)
    p1 = _make_conv(64)(z1, src, dst, zeros64)
    z2 = _tcmid64(degp, p1, z1, b1.reshape(1, -1), W2)
    p2 = _make_conv(32)(z2, src, dst, zeros32)
    z3 = _tcmid32(degp, p2, z2, b2.reshape(1, -1), W3)
    p3 = _make_conv(32)(z3, src, dst, zeros32)
    logits = _tc3(degp, p3, z3, b3.reshape(1, -1),
                  batch.reshape(1, -1), Wl, bl.reshape(1, -1))
    return logits
